# baseline (speedup 1.0000x reference)
import dataclasses
import functools

import jax
import jax.numpy as jnp
from jax import lax
from jax.experimental import pallas as pl
from jax.experimental.pallas import tpu as pltpu
from jax.experimental.pallas import tpu_sc as plsc

D_MODEL = 32
W = 128
NC, NS = 2, 16
NW = NC * NS
QUAD = 128
def kernel(x, table):
    b, s, f = x.shape
    n = b * s * f
    idx = jnp.transpose(x, (2, 1, 0)).reshape(n).astype(jnp.int32)
    vocab = table.shape[0]
    table4 = table.reshape(vocab // 4, QUAD)

    b_per_w = n // NW
    n_blocks = b_per_w // W
    bblocks = b // W
    d_out = f * D_MODEL

    mesh = plsc.VectorSubcoreMesh(core_axis_name="core", subcore_axis_name="subcore")
    cp = pltpu.CompilerParams()
    if "needs_layout_passes" in pltpu.CompilerParams.__dataclass_fields__:
        cp = dataclasses.replace(cp, needs_layout_passes=False)

    @functools.partial(
        pl.kernel,
        mesh=mesh,
        compiler_params=cp,
        out_type=jax.ShapeDtypeStruct((s, d_out, b), table.dtype),
        scratch_types=[
            pltpu.VMEM((b_per_w,), jnp.int32),
            pltpu.VMEM((b_per_w,), jnp.int32),
            pltpu.VMEM((W, QUAD), jnp.float32),
            pltpu.VMEM((W, QUAD), jnp.float32),
            pltpu.VMEM((W, QUAD), jnp.float32),
            pltpu.VMEM((W, QUAD), jnp.float32),
            pltpu.VMEM((D_MODEL, W), jnp.float32),
            pltpu.VMEM((D_MODEL, W), jnp.float32),
            pltpu.VMEM((D_MODEL, W), jnp.float32),
            pltpu.VMEM((D_MODEL, W), jnp.float32),
            pltpu.SemaphoreType.DMA,
            pltpu.SemaphoreType.DMA,
            pltpu.SemaphoreType.DMA,
            pltpu.SemaphoreType.DMA,
            pltpu.SemaphoreType.DMA,
            pltpu.SemaphoreType.DMA,
            pltpu.SemaphoreType.DMA,
            pltpu.SemaphoreType.DMA,
        ],
    )
    def gather_kernel(
        table4_hbm, idx_hbm, out_hbm,
        idx_v, gidx_v, rows0, rows1, rows2, rows3, out0, out1, out2, out3,
        gsem0, gsem1, gsem2, gsem3, osem0, osem1, osem2, osem3,
    ):
        rows_b = (rows0, rows1, rows2, rows3)
        out_b = (out0, out1, out2, out3)
        gsem_b = (gsem0, gsem1, gsem2, gsem3)
        osem_b = (osem0, osem1, osem2, osem3)
        wid = lax.axis_index("subcore") * NC + lax.axis_index("core")
        base = wid * b_per_w
        jbase = wid * n_blocks
        pltpu.sync_copy(idx_hbm.at[pl.ds(base, b_per_w)], idx_v)

        lane16 = lax.iota(jnp.int32, 16)

        @pl.loop(0, b_per_w, step=16)
        def _(i):
            gidx_v[pl.ds(i, 16)] = lax.shift_right_logical(idx_v[pl.ds(i, 16)], 2)

        def fire_gather(g, rows, gsem):
            pltpu.async_copy(
                table4_hbm.at[gidx_v.at[pl.ds(g * W, W)]], rows, gsem
            )

        def wait_gather(g, rows, gsem):
            pltpu.make_async_copy(
                table4_hbm.at[gidx_v.at[pl.ds(g * W, W)]], rows, gsem
            ).wait()

        def out_slice(g):
            j = jbase + g
            f0 = j // (s * bblocks)
            r = j - f0 * (s * bblocks)
            s0 = r // bblocks
            bblk = r - s0 * bblocks
            return out_hbm.at[s0, pl.ds(f0 * D_MODEL, D_MODEL), pl.ds(bblk * W, W)]

        def fire_store(g, out_v, osem):
            pltpu.async_copy(out_v, out_slice(g), osem)

        def wait_store(g, out_v, osem):
            pltpu.make_async_copy(out_v, out_slice(g), osem).wait()

        def select(g, rows, out_v):
            @pl.loop(0, W, step=16)
            def _(r0):
                t = idx_v[pl.ds(g * W + r0, 16)]
                cbase = (t & 3) * D_MODEL
                rvec = lane16 + r0
                for c0 in range(0, D_MODEL, 16):
                    vals = [
                        plsc.load_gather(rows, [rvec, cbase + (c0 + k)])
                        for k in range(16)
                    ]
                    for k in range(16):
                        out_v[c0 + k, pl.ds(r0, 16)] = vals[k]

        for i in range(4):
            fire_gather(i, rows_b[i], gsem_b[i])

        nb4 = (n_blocks // 4) * 4

        @pl.loop(0, nb4, step=4)
        def _(g):
            for i in range(4):
                wait_gather(g + i, rows_b[i], gsem_b[i])

                @pl.when(g + i >= 4)
                def _():
                    wait_store(g + i - 4, out_b[i], osem_b[i])

                select(g + i, rows_b[i], out_b[i])
                fire_store(g + i, out_b[i], osem_b[i])

                @pl.when(g + i + 4 < n_blocks)
                def _():
                    fire_gather(g + i + 4, rows_b[i], gsem_b[i])

        for j in range(nb4, n_blocks):
            i = j % 4
            wait_gather(j, rows_b[i], gsem_b[i])
            wait_store(j - 4, out_b[i], osem_b[i])
            select(j, rows_b[i], out_b[i])
            fire_store(j, out_b[i], osem_b[i])
        for j in range(n_blocks - 4, n_blocks):
            i = j % 4
            wait_store(j, out_b[i], osem_b[i])

    out3 = gather_kernel(table4, idx)
    return jnp.transpose(out3, (2, 0, 1))


# speedup vs baseline: 1.0071x; 1.0071x over previous
import dataclasses
import functools

import jax
import jax.numpy as jnp
from jax import lax
from jax.experimental import pallas as pl
from jax.experimental.pallas import tpu as pltpu
from jax.experimental.pallas import tpu_sc as plsc

D_MODEL = 32
W = 128
NC, NS = 2, 16
NW = NC * NS
QUAD = 128
def kernel(x, table):
    b, s, f = x.shape
    n = b * s * f
    idx = jnp.transpose(x, (2, 1, 0)).reshape(n).astype(jnp.int32)
    vocab = table.shape[0]
    table4 = table.reshape(vocab // 4, QUAD)

    b_per_w = n // NW
    n_blocks = b_per_w // W
    bblocks = b // W
    d_out = f * D_MODEL

    mesh = plsc.VectorSubcoreMesh(core_axis_name="core", subcore_axis_name="subcore")
    cp = pltpu.CompilerParams()
    if "needs_layout_passes" in pltpu.CompilerParams.__dataclass_fields__:
        cp = dataclasses.replace(cp, needs_layout_passes=False)

    @functools.partial(
        pl.kernel,
        mesh=mesh,
        compiler_params=cp,
        out_type=jax.ShapeDtypeStruct((s, d_out, b), table.dtype),
        scratch_types=[
            pltpu.VMEM((b_per_w,), jnp.int32),
            pltpu.VMEM((b_per_w,), jnp.int32),
            pltpu.VMEM((W, QUAD), jnp.float32),
            pltpu.VMEM((W, QUAD), jnp.float32),
            pltpu.VMEM((D_MODEL, W), jnp.float32),
            pltpu.VMEM((D_MODEL, W), jnp.float32),
            pltpu.SemaphoreType.DMA,
            pltpu.SemaphoreType.DMA,
            pltpu.SemaphoreType.DMA,
            pltpu.SemaphoreType.DMA,
        ],
    )
    def gather_kernel(
        table4_hbm, idx_hbm, out_hbm,
        idx_v, gidx_v, rows0, rows1, out0, out1,
        gsem0, gsem1, osem0, osem1,
    ):
        wid = lax.axis_index("subcore") * NC + lax.axis_index("core")
        base = wid * b_per_w
        jbase = wid * n_blocks
        pltpu.sync_copy(idx_hbm.at[pl.ds(base, b_per_w)], idx_v)

        lane16 = lax.iota(jnp.int32, 16)

        @pl.loop(0, b_per_w, step=16)
        def _(i):
            gidx_v[pl.ds(i, 16)] = lax.shift_right_logical(idx_v[pl.ds(i, 16)], 2)

        def fire_gather(g, rows, gsem):
            pltpu.async_copy(
                table4_hbm.at[gidx_v.at[pl.ds(g * W, W)]], rows, gsem
            )

        def wait_gather(g, rows, gsem):
            pltpu.make_async_copy(
                table4_hbm.at[gidx_v.at[pl.ds(g * W, W)]], rows, gsem
            ).wait()

        def out_slice(g):
            j = jbase + g
            f0 = j // (s * bblocks)
            r = j - f0 * (s * bblocks)
            s0 = r // bblocks
            bblk = r - s0 * bblocks
            return out_hbm.at[s0, pl.ds(f0 * D_MODEL, D_MODEL), pl.ds(bblk * W, W)]

        def fire_store(g, out_v, osem):
            pltpu.async_copy(out_v, out_slice(g), osem)

        def wait_store(g, out_v, osem):
            pltpu.make_async_copy(out_v, out_slice(g), osem).wait()

        def select(g, rows, out_v):
            @pl.loop(0, W, step=16)
            def _(r0):
                t = idx_v[pl.ds(g * W + r0, 16)]
                cbase = (t & 3) * D_MODEL
                rvec = lane16 + r0
                for c0 in range(0, D_MODEL, 16):
                    vals = [
                        plsc.load_gather(rows, [rvec, cbase + (c0 + k)])
                        for k in range(16)
                    ]
                    for k in range(16):
                        out_v[c0 + k, pl.ds(r0, 16)] = vals[k]

        fire_gather(0, rows0, gsem0)
        fire_gather(1, rows1, gsem1)

        @pl.loop(0, n_blocks, step=2)
        def _(g):
            wait_gather(g, rows0, gsem0)

            @pl.when(g > 0)
            def _():
                wait_store(g - 2, out0, osem0)

            select(g, rows0, out0)
            fire_store(g, out0, osem0)

            @pl.when(g + 2 < n_blocks)
            def _():
                fire_gather(g + 2, rows0, gsem0)

            wait_gather(g + 1, rows1, gsem1)

            @pl.when(g > 0)
            def _():
                wait_store(g - 1, out1, osem1)

            select(g + 1, rows1, out1)
            fire_store(g + 1, out1, osem1)

            @pl.when(g + 3 < n_blocks)
            def _():
                fire_gather(g + 3, rows1, gsem1)

        wait_store(n_blocks - 2, out0, osem0)
        wait_store(n_blocks - 1, out1, osem1)

    out3 = gather_kernel(table4, idx)
    return jnp.transpose(out3, (2, 0, 1))
